# baseline (speedup 1.0000x reference)
import functools

import jax
import jax.numpy as jnp
from jax import lax
from jax.experimental import pallas as pl
from jax.experimental.pallas import tpu as pltpu
from jax.experimental.pallas import tpu_sc as plsc

N = 10000
D = 256
H = 256
C = 10
E = 160000

NC = 2
NS = 16
HH = 128
NPAD = 10240
EPAD = 163840
RPT = NPAD // NS
GB2 = 64
EB_AGG = EPAD // NS // GB2
EB_DEG = EPAD // (NC * NS) // 128
BLK = 256
NBLK = NPAD // BLK
NBUF = 5

_MESH = plsc.VectorSubcoreMesh(
    core_axis_name="c", subcore_axis_name="s", num_cores=NC, num_subcores=NS
)


@functools.partial(
    pl.kernel,
    out_type=jax.ShapeDtypeStruct((NC, NPAD), jnp.float32),
    mesh=_MESH,
    scratch_types=[
        pltpu.VMEM((EB_DEG, 128), jnp.int32),
        pltpu.VMEM((128,), jnp.float32),
        pltpu.VMEM_SHARED((NPAD,), jnp.float32),
    ],
)
def _deg_kernel(dst_hbm, z1_hbm, degp_hbm, dstv, ones_v, deg_sh):
    c = lax.axis_index("c")
    s = lax.axis_index("s")
    wid = c * NS + s
    pltpu.sync_copy(dst_hbm.at[wid], dstv)
    for i in range(8):
        ones_v[pl.ds(i * 16, 16)] = jnp.ones((16,), jnp.float32)
    pltpu.sync_copy(z1_hbm.at[pl.ds(s * RPT, RPT)], deg_sh.at[pl.ds(s * RPT, RPT)])
    plsc.subcore_barrier()

    @pl.loop(0, EB_DEG)
    def _(j):
        pltpu.sync_copy(ones_v, deg_sh.at[dstv.at[j]], add=True)

    plsc.subcore_barrier()

    @pl.when(s == 0)
    def _():
        pltpu.sync_copy(deg_sh, degp_hbm.at[c])


@functools.partial(
    pl.kernel,
    out_type=(
        jax.ShapeDtypeStruct((NPAD, HH), jnp.float32),
        jax.ShapeDtypeStruct((NPAD, HH), jnp.float32),
    ),
    mesh=_MESH,
    scratch_types=[
        pltpu.VMEM((GB2,), jnp.int32),
        pltpu.VMEM((GB2,), jnp.int32),
        pltpu.VMEM((GB2,), jnp.int32),
        pltpu.VMEM((GB2,), jnp.int32),
        pltpu.VMEM((GB2,), jnp.int32),
        pltpu.VMEM((GB2,), jnp.int32),
        pltpu.VMEM((GB2,), jnp.int32),
        pltpu.VMEM((GB2,), jnp.int32),
        pltpu.VMEM((GB2,), jnp.int32),
        pltpu.VMEM((GB2,), jnp.int32),
        pltpu.VMEM((GB2, HH), jnp.float32),
        pltpu.VMEM((GB2, HH), jnp.float32),
        pltpu.VMEM((GB2, HH), jnp.float32),
        pltpu.VMEM((GB2, HH), jnp.float32),
        pltpu.VMEM((GB2, HH), jnp.float32),
        pltpu.VMEM_SHARED((NPAD, HH), jnp.float32),
        pltpu.SemaphoreType.DMA,
        pltpu.SemaphoreType.DMA,
        pltpu.SemaphoreType.DMA,
        pltpu.SemaphoreType.DMA,
        pltpu.SemaphoreType.DMA,
        pltpu.SemaphoreType.DMA,
        pltpu.SemaphoreType.DMA,
        pltpu.SemaphoreType.DMA,
        pltpu.SemaphoreType.DMA,
        pltpu.SemaphoreType.DMA,
        pltpu.SemaphoreType.DMA,
        pltpu.SemaphoreType.DMA,
        pltpu.SemaphoreType.DMA,
        pltpu.SemaphoreType.DMA,
        pltpu.SemaphoreType.DMA,
    ],
)
def _agg_kernel(hsa_hbm, hsb_hbm, src_hbm, dst_hbm, z2_hbm,
                acca_hbm, accb_hbm,
                i0, i1, i2, i3, i4, d0, d1, d2, d3, d4,
                r0, r1, r2, r3, r4, acc_sh,
                g0, g1, g2, g3, g4, s0, s1, s2, s3, s4, t0, t1, t2, t3, t4):
    islot = [i0, i1, i2, i3, i4]
    dslot = [d0, d1, d2, d3, d4]
    rows = [r0, r1, r2, r3, r4]
    gsem = [g0, g1, g2, g3, g4]
    isem = [s0, s1, s2, s3, s4]
    dsem = [t0, t1, t2, t3, t4]
    c = lax.axis_index("c")
    s = lax.axis_index("s")
    pltpu.sync_copy(z2_hbm.at[pl.ds(s * RPT, RPT)], acc_sh.at[pl.ds(s * RPT, RPT)])
    plsc.subcore_barrier()

    def _srcload(j, b):
        pltpu.async_copy(src_hbm.at[s, j], islot[b], isem[b])

    def _dstload(j, b):
        pltpu.async_copy(dst_hbm.at[s, j], dslot[b], dsem[b])

    def _gather(b):
        @pl.when(c == 0)
        def _():
            pltpu.async_copy(hsa_hbm.at[islot[b]], rows[b], gsem[b])

        @pl.when(c == 1)
        def _():
            pltpu.async_copy(hsb_hbm.at[islot[b]], rows[b], gsem[b])

    for b in range(NBUF):
        _srcload(b, b)
        _dstload(b, b)
    for b in range(NBUF):
        pltpu.make_async_copy(src_hbm.at[s, b], islot[b], isem[b]).wait()
        _gather(b)

    @pl.loop(0, EB_AGG, step=NBUF)
    def _(g):
        for b in range(NBUF):
            j = g + b
            pltpu.make_async_copy(hsa_hbm.at[islot[b]], rows[b], gsem[b]).wait()
            nj = j + NBUF
            nj = jnp.where(nj < EB_AGG, nj, nj - EB_AGG)
            _srcload(nj, b)
            pltpu.make_async_copy(dst_hbm.at[s, j], dslot[b], dsem[b]).wait()
            pltpu.sync_copy(rows[b], acc_sh.at[dslot[b]], add=True)
            _dstload(nj, b)
            pltpu.make_async_copy(src_hbm.at[s, j], islot[b], isem[b]).wait()
            _gather(b)

    for b in range(NBUF):
        pltpu.make_async_copy(hsa_hbm.at[islot[b]], rows[b], gsem[b]).wait()
        pltpu.make_async_copy(dst_hbm.at[s, 0], dslot[b], dsem[b]).wait()

    plsc.subcore_barrier()

    @pl.when(c == 0)
    def _():
        pltpu.sync_copy(acc_sh.at[pl.ds(s * RPT, RPT)],
                        acca_hbm.at[pl.ds(s * RPT, RPT)])

    @pl.when(c == 1)
    def _():
        pltpu.sync_copy(acc_sh.at[pl.ds(s * RPT, RPT)],
                        accb_hbm.at[pl.ds(s * RPT, RPT)])


def _mm1_body(x_ref, w_ref, d0_ref, d1_ref, hsa_ref, hsb_ref, dinv_ref):
    i = pl.program_id(0)
    deg = d0_ref[...] + d1_ref[...] + 1.0
    rows = i * BLK + lax.broadcasted_iota(jnp.int32, (BLK, 1), 0)
    dinv = jnp.where(rows < N, lax.rsqrt(deg), 0.0)
    dinv_ref[...] = dinv
    h = jnp.dot(x_ref[...], w_ref[...], preferred_element_type=jnp.float32)
    hs = h * dinv
    hsa_ref[...] = hs[:, :HH]
    hsb_ref[...] = hs[:, HH:]


def _mm1(xp, W1, d0, d1):
    return pl.pallas_call(
        _mm1_body,
        grid=(NBLK,),
        in_specs=[
            pl.BlockSpec((BLK, D), lambda i: (i, 0)),
            pl.BlockSpec((D, H), lambda i: (0, 0)),
            pl.BlockSpec((BLK, 1), lambda i: (i, 0)),
            pl.BlockSpec((BLK, 1), lambda i: (i, 0)),
        ],
        out_specs=[
            pl.BlockSpec((BLK, HH), lambda i: (i, 0)),
            pl.BlockSpec((BLK, HH), lambda i: (i, 0)),
            pl.BlockSpec((BLK, 1), lambda i: (i, 0)),
        ],
        out_shape=[
            jax.ShapeDtypeStruct((NPAD, HH), jnp.float32),
            jax.ShapeDtypeStruct((NPAD, HH), jnp.float32),
            jax.ShapeDtypeStruct((NPAD, 1), jnp.float32),
        ],
    )(xp, W1, d0, d1)


def _mm2_body(acca_ref, accb_ref, hsa_ref, hsb_ref, dinv_ref, b1_ref, w2_ref,
              outa_ref, outb_ref):
    dinv = dinv_ref[...]
    agg = jnp.concatenate(
        [acca_ref[...] + hsa_ref[...], accb_ref[...] + hsb_ref[...]], axis=1)
    t = jnp.maximum(agg * dinv + b1_ref[...], 0.0)
    h2 = jnp.dot(t, w2_ref[...], preferred_element_type=jnp.float32)
    hs2 = h2 * dinv
    outa_ref[...] = hs2[:, :HH]
    outb_ref[...] = hs2[:, HH:]


def _mm2(accA, accB, hsA, hsB, dinv, b1r, W2):
    return pl.pallas_call(
        _mm2_body,
        grid=(NBLK,),
        in_specs=[
            pl.BlockSpec((BLK, HH), lambda i: (i, 0)),
            pl.BlockSpec((BLK, HH), lambda i: (i, 0)),
            pl.BlockSpec((BLK, HH), lambda i: (i, 0)),
            pl.BlockSpec((BLK, HH), lambda i: (i, 0)),
            pl.BlockSpec((BLK, 1), lambda i: (i, 0)),
            pl.BlockSpec((1, H), lambda i: (0, 0)),
            pl.BlockSpec((H, H), lambda i: (0, 0)),
        ],
        out_specs=[
            pl.BlockSpec((BLK, HH), lambda i: (i, 0)),
            pl.BlockSpec((BLK, HH), lambda i: (i, 0)),
        ],
        out_shape=[
            jax.ShapeDtypeStruct((NPAD, HH), jnp.float32),
            jax.ShapeDtypeStruct((NPAD, HH), jnp.float32),
        ],
    )(accA, accB, hsA, hsB, dinv, b1r, W2)


def _mm3_body(acca_ref, accb_ref, hsa_ref, hsb_ref, dinv_ref, b2_ref, wc_ref,
              bc_ref, out_ref, acc_scratch):
    i = pl.program_id(0)
    dinv = dinv_ref[...]
    agg = jnp.concatenate(
        [acca_ref[...] + hsa_ref[...], accb_ref[...] + hsb_ref[...]], axis=1)
    o = jnp.maximum(agg * dinv + b2_ref[...], 0.0)
    rows = i * BLK + lax.broadcasted_iota(jnp.int32, (BLK, 1), 0)
    o = jnp.where(rows < N, o, 0.0)
    part = jnp.sum(o, axis=0, keepdims=True)

    @pl.when(i == 0)
    def _():
        acc_scratch[...] = part

    @pl.when(i > 0)
    def _():
        acc_scratch[...] = acc_scratch[...] + part

    @pl.when(i == pl.num_programs(0) - 1)
    def _():
        z = acc_scratch[...] * (1.0 / N)
        out_ref[...] = (
            jnp.dot(z, wc_ref[...], preferred_element_type=jnp.float32)
            + bc_ref[...])


def _mm3(accA, accB, hsA, hsB, dinv, b2r, Wcp, bcp):
    return pl.pallas_call(
        _mm3_body,
        grid=(NBLK,),
        in_specs=[
            pl.BlockSpec((BLK, HH), lambda i: (i, 0)),
            pl.BlockSpec((BLK, HH), lambda i: (i, 0)),
            pl.BlockSpec((BLK, HH), lambda i: (i, 0)),
            pl.BlockSpec((BLK, HH), lambda i: (i, 0)),
            pl.BlockSpec((BLK, 1), lambda i: (i, 0)),
            pl.BlockSpec((1, H), lambda i: (0, 0)),
            pl.BlockSpec((H, 128), lambda i: (0, 0)),
            pl.BlockSpec((1, 128), lambda i: (0, 0)),
        ],
        out_specs=pl.BlockSpec((1, 128), lambda i: (0, 0)),
        out_shape=jax.ShapeDtypeStruct((1, 128), jnp.float32),
        scratch_shapes=[pltpu.VMEM((1, H), jnp.float32)],
    )(accA, accB, hsA, hsB, dinv, b2r, Wcp, bcp)


def kernel(x, edge_index, W1, b1, W2, b2, Wc, bc):
    src = edge_index[0]
    dst = edge_index[1]
    idx_pad = jnp.full((EPAD - E,), N, jnp.int32)
    srcp = jnp.concatenate([src, idx_pad]).reshape(NS, EB_AGG, GB2)
    dstp = jnp.concatenate([dst, idx_pad])
    dst16 = dstp.reshape(NS, EB_AGG, GB2)
    dst32 = dstp.reshape(NC * NS, EB_DEG, 128)

    xp = jnp.concatenate([x, jnp.zeros((NPAD - N, D), jnp.float32)], axis=0)
    z1 = jnp.zeros((NPAD,), jnp.float32)
    z2 = jnp.zeros((NPAD, HH), jnp.float32)
    b1r = b1.reshape(1, H)
    b2r = b2.reshape(1, H)
    Wcp = jnp.concatenate([Wc, jnp.zeros((H, 128 - C), jnp.float32)], axis=1)
    bcp = jnp.concatenate([bc, jnp.zeros((128 - C,), jnp.float32)]).reshape(1, 128)

    degp = _deg_kernel(dst32, z1)
    d0 = degp[0].reshape(NPAD, 1)
    d1 = degp[1].reshape(NPAD, 1)

    hsA, hsB, dinv = _mm1(xp, W1, d0, d1)
    accA, accB = _agg_kernel(hsA, hsB, srcp, dst16, z2)
    hs2A, hs2B = _mm2(accA, accB, hsA, hsB, dinv, b1r, W2)
    acc2A, acc2B = _agg_kernel(hs2A, hs2B, srcp, dst16, z2)
    logits = _mm3(acc2A, acc2B, hs2A, hs2B, dinv, b2r, Wcp, bcp)
    return logits[:, :C].reshape(1, 1, C)


# speedup vs baseline: 1.0005x; 1.0005x over previous
import functools

import jax
import jax.numpy as jnp
from jax import lax
from jax.experimental import pallas as pl
from jax.experimental.pallas import tpu as pltpu
from jax.experimental.pallas import tpu_sc as plsc

N = 10000
D = 256
H = 256
C = 10
E = 160000

NC = 2
NS = 16
HH = 128
NPAD = 10240
EPAD = 163840
RPT = NPAD // NS
GB2 = 64
EB_AGG = EPAD // NS // GB2
EB_DEG = EPAD // (NC * NS) // 128
BLK = 256
NBLK = NPAD // BLK
NBUF = 4

_MESH = plsc.VectorSubcoreMesh(
    core_axis_name="c", subcore_axis_name="s", num_cores=NC, num_subcores=NS
)


@functools.partial(
    pl.kernel,
    out_type=jax.ShapeDtypeStruct((NC, NPAD), jnp.float32),
    mesh=_MESH,
    scratch_types=[
        pltpu.VMEM((EB_DEG, 128), jnp.int32),
        pltpu.VMEM((128,), jnp.float32),
        pltpu.VMEM_SHARED((NPAD,), jnp.float32),
    ],
)
def _deg_kernel(dst_hbm, z1_hbm, degp_hbm, dstv, ones_v, deg_sh):
    c = lax.axis_index("c")
    s = lax.axis_index("s")
    wid = c * NS + s
    pltpu.sync_copy(dst_hbm.at[wid], dstv)
    for i in range(8):
        ones_v[pl.ds(i * 16, 16)] = jnp.ones((16,), jnp.float32)
    pltpu.sync_copy(z1_hbm.at[pl.ds(s * RPT, RPT)], deg_sh.at[pl.ds(s * RPT, RPT)])
    plsc.subcore_barrier()

    @pl.loop(0, EB_DEG)
    def _(j):
        pltpu.sync_copy(ones_v, deg_sh.at[dstv.at[j]], add=True)

    plsc.subcore_barrier()

    @pl.when(s == 0)
    def _():
        pltpu.sync_copy(deg_sh, degp_hbm.at[c])


@functools.partial(
    pl.kernel,
    out_type=(
        jax.ShapeDtypeStruct((NPAD, HH), jnp.float32),
        jax.ShapeDtypeStruct((NPAD, HH), jnp.float32),
    ),
    mesh=_MESH,
    scratch_types=[
        pltpu.VMEM((GB2,), jnp.int32),
        pltpu.VMEM((GB2,), jnp.int32),
        pltpu.VMEM((GB2,), jnp.int32),
        pltpu.VMEM((GB2,), jnp.int32),
        pltpu.VMEM((GB2,), jnp.int32),
        pltpu.VMEM((GB2,), jnp.int32),
        pltpu.VMEM((GB2,), jnp.int32),
        pltpu.VMEM((GB2,), jnp.int32),
        pltpu.VMEM((GB2, HH), jnp.float32),
        pltpu.VMEM((GB2, HH), jnp.float32),
        pltpu.VMEM((GB2, HH), jnp.float32),
        pltpu.VMEM((GB2, HH), jnp.float32),
        pltpu.VMEM_SHARED((NPAD, HH), jnp.float32),
        pltpu.SemaphoreType.DMA,
        pltpu.SemaphoreType.DMA,
        pltpu.SemaphoreType.DMA,
        pltpu.SemaphoreType.DMA,
        pltpu.SemaphoreType.DMA,
        pltpu.SemaphoreType.DMA,
        pltpu.SemaphoreType.DMA,
        pltpu.SemaphoreType.DMA,
        pltpu.SemaphoreType.DMA,
        pltpu.SemaphoreType.DMA,
        pltpu.SemaphoreType.DMA,
        pltpu.SemaphoreType.DMA,
    ],
)
def _agg_kernel(hsa_hbm, hsb_hbm, src_hbm, dst_hbm, z2_hbm,
                acca_hbm, accb_hbm,
                i0, i1, i2, i3, d0, d1, d2, d3, r0, r1, r2, r3, acc_sh,
                g0, g1, g2, g3, s0, s1, s2, s3, t0, t1, t2, t3):
    islot = [i0, i1, i2, i3]
    dslot = [d0, d1, d2, d3]
    rows = [r0, r1, r2, r3]
    gsem = [g0, g1, g2, g3]
    isem = [s0, s1, s2, s3]
    dsem = [t0, t1, t2, t3]
    c = lax.axis_index("c")
    s = lax.axis_index("s")
    pltpu.sync_copy(z2_hbm.at[pl.ds(s * RPT, RPT)], acc_sh.at[pl.ds(s * RPT, RPT)])
    plsc.subcore_barrier()

    def _srcload(j, b):
        pltpu.async_copy(src_hbm.at[s, j], islot[b], isem[b])

    def _dstload(j, b):
        pltpu.async_copy(dst_hbm.at[s, j], dslot[b], dsem[b])

    def _gather(b):
        @pl.when(c == 0)
        def _():
            pltpu.async_copy(hsa_hbm.at[islot[b]], rows[b], gsem[b])

        @pl.when(c == 1)
        def _():
            pltpu.async_copy(hsb_hbm.at[islot[b]], rows[b], gsem[b])

    for b in range(NBUF):
        _srcload(b, b)
        _dstload(b, b)
    for b in range(NBUF):
        pltpu.make_async_copy(src_hbm.at[s, b], islot[b], isem[b]).wait()
        _gather(b)

    @pl.loop(0, EB_AGG, step=NBUF)
    def _(g):
        for b in range(NBUF):
            j = g + b
            pltpu.make_async_copy(hsa_hbm.at[islot[b]], rows[b], gsem[b]).wait()
            nj = j + NBUF
            nj = jnp.where(nj < EB_AGG, nj, nj - EB_AGG)
            _srcload(nj, b)
            pltpu.make_async_copy(dst_hbm.at[s, j], dslot[b], dsem[b]).wait()
            pltpu.sync_copy(rows[b], acc_sh.at[dslot[b]], add=True)
            _dstload(nj, b)
            pltpu.make_async_copy(src_hbm.at[s, j], islot[b], isem[b]).wait()
            _gather(b)

    for b in range(NBUF):
        pltpu.make_async_copy(hsa_hbm.at[islot[b]], rows[b], gsem[b]).wait()
        pltpu.make_async_copy(dst_hbm.at[s, 0], dslot[b], dsem[b]).wait()

    plsc.subcore_barrier()

    @pl.when(c == 0)
    def _():
        pltpu.sync_copy(acc_sh.at[pl.ds(s * RPT, RPT)],
                        acca_hbm.at[pl.ds(s * RPT, RPT)])

    @pl.when(c == 1)
    def _():
        pltpu.sync_copy(acc_sh.at[pl.ds(s * RPT, RPT)],
                        accb_hbm.at[pl.ds(s * RPT, RPT)])


def _mm1_body(x_ref, w_ref, d0_ref, d1_ref, hsa_ref, hsb_ref, dinv_ref):
    i = pl.program_id(0)
    deg = d0_ref[...] + d1_ref[...] + 1.0
    rows = i * BLK + lax.broadcasted_iota(jnp.int32, (BLK, 1), 0)
    dinv = jnp.where(rows < N, lax.rsqrt(deg), 0.0)
    dinv_ref[...] = dinv
    h = jnp.dot(x_ref[...], w_ref[...], preferred_element_type=jnp.float32)
    hs = h * dinv
    hsa_ref[...] = hs[:, :HH]
    hsb_ref[...] = hs[:, HH:]


def _mm1(xp, W1, d0, d1):
    return pl.pallas_call(
        _mm1_body,
        grid=(NBLK,),
        in_specs=[
            pl.BlockSpec((BLK, D), lambda i: (i, 0)),
            pl.BlockSpec((D, H), lambda i: (0, 0)),
            pl.BlockSpec((BLK, 1), lambda i: (i, 0)),
            pl.BlockSpec((BLK, 1), lambda i: (i, 0)),
        ],
        out_specs=[
            pl.BlockSpec((BLK, HH), lambda i: (i, 0)),
            pl.BlockSpec((BLK, HH), lambda i: (i, 0)),
            pl.BlockSpec((BLK, 1), lambda i: (i, 0)),
        ],
        out_shape=[
            jax.ShapeDtypeStruct((NPAD, HH), jnp.float32),
            jax.ShapeDtypeStruct((NPAD, HH), jnp.float32),
            jax.ShapeDtypeStruct((NPAD, 1), jnp.float32),
        ],
    )(xp, W1, d0, d1)


def _mm2_body(acca_ref, accb_ref, hsa_ref, hsb_ref, dinv_ref, b1_ref, w2_ref,
              outa_ref, outb_ref):
    dinv = dinv_ref[...]
    agg = jnp.concatenate(
        [acca_ref[...] + hsa_ref[...], accb_ref[...] + hsb_ref[...]], axis=1)
    t = jnp.maximum(agg * dinv + b1_ref[...], 0.0)
    h2 = jnp.dot(t, w2_ref[...], preferred_element_type=jnp.float32)
    hs2 = h2 * dinv
    outa_ref[...] = hs2[:, :HH]
    outb_ref[...] = hs2[:, HH:]


def _mm2(accA, accB, hsA, hsB, dinv, b1r, W2):
    return pl.pallas_call(
        _mm2_body,
        grid=(NBLK,),
        in_specs=[
            pl.BlockSpec((BLK, HH), lambda i: (i, 0)),
            pl.BlockSpec((BLK, HH), lambda i: (i, 0)),
            pl.BlockSpec((BLK, HH), lambda i: (i, 0)),
            pl.BlockSpec((BLK, HH), lambda i: (i, 0)),
            pl.BlockSpec((BLK, 1), lambda i: (i, 0)),
            pl.BlockSpec((1, H), lambda i: (0, 0)),
            pl.BlockSpec((H, H), lambda i: (0, 0)),
        ],
        out_specs=[
            pl.BlockSpec((BLK, HH), lambda i: (i, 0)),
            pl.BlockSpec((BLK, HH), lambda i: (i, 0)),
        ],
        out_shape=[
            jax.ShapeDtypeStruct((NPAD, HH), jnp.float32),
            jax.ShapeDtypeStruct((NPAD, HH), jnp.float32),
        ],
    )(accA, accB, hsA, hsB, dinv, b1r, W2)


def _mm3_body(acca_ref, accb_ref, hsa_ref, hsb_ref, dinv_ref, b2_ref, wc_ref,
              bc_ref, out_ref, acc_scratch):
    i = pl.program_id(0)
    dinv = dinv_ref[...]
    agg = jnp.concatenate(
        [acca_ref[...] + hsa_ref[...], accb_ref[...] + hsb_ref[...]], axis=1)
    o = jnp.maximum(agg * dinv + b2_ref[...], 0.0)
    rows = i * BLK + lax.broadcasted_iota(jnp.int32, (BLK, 1), 0)
    o = jnp.where(rows < N, o, 0.0)
    part = jnp.sum(o, axis=0, keepdims=True)

    @pl.when(i == 0)
    def _():
        acc_scratch[...] = part

    @pl.when(i > 0)
    def _():
        acc_scratch[...] = acc_scratch[...] + part

    @pl.when(i == pl.num_programs(0) - 1)
    def _():
        z = acc_scratch[...] * (1.0 / N)
        out_ref[...] = (
            jnp.dot(z, wc_ref[...], preferred_element_type=jnp.float32)
            + bc_ref[...])


def _mm3(accA, accB, hsA, hsB, dinv, b2r, Wcp, bcp):
    return pl.pallas_call(
        _mm3_body,
        grid=(NBLK,),
        in_specs=[
            pl.BlockSpec((BLK, HH), lambda i: (i, 0)),
            pl.BlockSpec((BLK, HH), lambda i: (i, 0)),
            pl.BlockSpec((BLK, HH), lambda i: (i, 0)),
            pl.BlockSpec((BLK, HH), lambda i: (i, 0)),
            pl.BlockSpec((BLK, 1), lambda i: (i, 0)),
            pl.BlockSpec((1, H), lambda i: (0, 0)),
            pl.BlockSpec((H, 128), lambda i: (0, 0)),
            pl.BlockSpec((1, 128), lambda i: (0, 0)),
        ],
        out_specs=pl.BlockSpec((1, 128), lambda i: (0, 0)),
        out_shape=jax.ShapeDtypeStruct((1, 128), jnp.float32),
        scratch_shapes=[pltpu.VMEM((1, H), jnp.float32)],
    )(accA, accB, hsA, hsB, dinv, b2r, Wcp, bcp)


def kernel(x, edge_index, W1, b1, W2, b2, Wc, bc):
    src = edge_index[0]
    dst = edge_index[1]
    idx_pad = jnp.full((EPAD - E,), N, jnp.int32)
    srcp = jnp.concatenate([src, idx_pad]).reshape(NS, EB_AGG, GB2)
    dstp = jnp.concatenate([dst, idx_pad])
    dst16 = dstp.reshape(NS, EB_AGG, GB2)
    dst32 = dstp.reshape(NC * NS, EB_DEG, 128)

    xp = jnp.concatenate([x, jnp.zeros((NPAD - N, D), jnp.float32)], axis=0)
    z1 = jnp.zeros((NPAD,), jnp.float32)
    z2 = jnp.zeros((NPAD, HH), jnp.float32)
    b1r = b1.reshape(1, H)
    b2r = b2.reshape(1, H)
    Wcp = jnp.concatenate([Wc, jnp.zeros((H, 128 - C), jnp.float32)], axis=1)
    bcp = jnp.concatenate([bc, jnp.zeros((128 - C,), jnp.float32)]).reshape(1, 128)

    degp = _deg_kernel(dst32, z1)
    d0 = degp[0].reshape(NPAD, 1)
    d1 = degp[1].reshape(NPAD, 1)

    hsA, hsB, dinv = _mm1(xp, W1, d0, d1)
    accA, accB = _agg_kernel(hsA, hsB, srcp, dst16, z2)
    hs2A, hs2B = _mm2(accA, accB, hsA, hsB, dinv, b1r, W2)
    acc2A, acc2B = _agg_kernel(hs2A, hs2B, srcp, dst16, z2)
    logits = _mm3(acc2A, acc2B, hs2A, hs2B, dinv, b2r, Wcp, bcp)
    return logits[:, :C].reshape(1, 1, C)
